# baseline (speedup 1.0000x reference)
import functools
import jax
import jax.numpy as jnp
from jax import lax
from jax.experimental import pallas as pl
from jax.experimental.pallas import tpu as pltpu
from jax.experimental.pallas import tpu_sc as plsc

NUM_CLASSES = 1000
ROWS = 4096
COLS = 26
PCOLS = 32


def _sc_onehot(xp_hbm, zeros_hbm, out_hbm, xrow_v, plane_v, sem):
    nc = 2
    ns = 16
    nw = nc * ns
    planes_per_w = ROWS // nw
    wid = lax.axis_index("s") * nc + lax.axis_index("c")

    pltpu.sync_copy(zeros_hbm, plane_v)

    ones = jnp.full((16,), 1.0, dtype=jnp.float32)
    zeros16 = jnp.zeros((16,), dtype=jnp.float32)
    row_lo = lax.iota(jnp.int32, 16)
    row_hi = row_lo + 16
    hi_mask = row_lo < (COLS - 16)

    def body(i, carry):
        p = wid * planes_per_w + i
        pltpu.sync_copy(xp_hbm.at[p], xrow_v)
        idx_lo = xrow_v[pl.ds(0, 16)]
        idx_hi = xrow_v[pl.ds(16, 16)]
        plsc.store_scatter(plane_v, [row_lo, idx_lo], ones)
        plsc.store_scatter(plane_v, [row_hi, idx_hi], ones, mask=hi_mask)
        pltpu.sync_copy(plane_v, out_hbm.at[p])
        plsc.store_scatter(plane_v, [row_lo, idx_lo], zeros16)
        plsc.store_scatter(plane_v, [row_hi, idx_hi], zeros16, mask=hi_mask)
        return carry

    lax.fori_loop(0, planes_per_w, body, 0)


def kernel(x):
    xi = x.astype(jnp.int32)
    xp = jnp.pad(xi, ((0, 0), (0, PCOLS - COLS)), constant_values=0)
    zeros = jnp.zeros((COLS, NUM_CLASSES), jnp.float32)
    mesh = plsc.VectorSubcoreMesh(core_axis_name="c", subcore_axis_name="s")
    k = functools.partial(
        pl.kernel,
        mesh=mesh,
        out_type=jax.ShapeDtypeStruct((ROWS, COLS, NUM_CLASSES), jnp.float32),
        scratch_types=[
            pltpu.VMEM((PCOLS,), jnp.int32),
            pltpu.VMEM((COLS, NUM_CLASSES), jnp.float32),
            pltpu.SemaphoreType.DMA,
        ],
        compiler_params=pltpu.CompilerParams(needs_layout_passes=False),
    )(_sc_onehot)
    return k(xp, zeros)


# speedup vs baseline: 1.0988x; 1.0988x over previous
import functools
import jax
import jax.numpy as jnp
from jax import lax
from jax.experimental import pallas as pl
from jax.experimental.pallas import tpu as pltpu
from jax.experimental.pallas import tpu_sc as plsc

NUM_CLASSES = 1000
ROWS = 4096
COLS = 26
PCOLS = 32
NBUF = 2


def _sc_onehot(xp_hbm, zeros_hbm, out_hbm, xrow_v, plane_v, sems):
    nc = 2
    ns = 16
    nw = nc * ns
    planes_per_w = ROWS // nw
    ngroups = planes_per_w // NBUF
    wid = lax.axis_index("s") * nc + lax.axis_index("c")
    base = wid * planes_per_w

    for b in range(NBUF):
        pltpu.sync_copy(zeros_hbm, plane_v.at[b])

    ones = jnp.full((16,), 1.0, dtype=jnp.float32)
    zeros16 = jnp.zeros((16,), dtype=jnp.float32)
    row_lo = lax.iota(jnp.int32, 16)
    row_hi = row_lo + 16
    hi_mask = row_lo < (COLS - 16)

    def scatter(b, vals):
        idx_lo = xrow_v[b, pl.ds(0, 16)]
        idx_hi = xrow_v[b, pl.ds(16, 16)]
        plsc.store_scatter(plane_v.at[b], [row_lo, idx_lo], vals)
        plsc.store_scatter(plane_v.at[b], [row_hi, idx_hi], vals, mask=hi_mask)

    def group(g, carry):
        for b in range(NBUF):
            i = g * NBUF + b
            p = base + i

            @pl.when(g > 0)
            def _recycle():
                pltpu.make_async_copy(
                    plane_v.at[b], out_hbm.at[p - NBUF], sems.at[b]
                ).wait()
                scatter(b, zeros16)

            pltpu.sync_copy(xp_hbm.at[p], xrow_v.at[b])
            scatter(b, ones)
            pltpu.make_async_copy(
                plane_v.at[b], out_hbm.at[p], sems.at[b]
            ).start()
        return carry

    lax.fori_loop(0, ngroups, group, 0)

    for b in range(NBUF):
        p = base + (ngroups - 1) * NBUF + b
        pltpu.make_async_copy(
            plane_v.at[b], out_hbm.at[p], sems.at[b]
        ).wait()


def kernel(x):
    xi = x.astype(jnp.int32)
    xp = jnp.pad(xi, ((0, 0), (0, PCOLS - COLS)), constant_values=0)
    zeros = jnp.zeros((COLS, NUM_CLASSES), jnp.float32)
    mesh = plsc.VectorSubcoreMesh(core_axis_name="c", subcore_axis_name="s")
    k = functools.partial(
        pl.kernel,
        mesh=mesh,
        out_type=jax.ShapeDtypeStruct((ROWS, COLS, NUM_CLASSES), jnp.float32),
        scratch_types=[
            pltpu.VMEM((NBUF, PCOLS), jnp.int32),
            pltpu.VMEM((NBUF, COLS, NUM_CLASSES), jnp.float32),
            pltpu.SemaphoreType.DMA((NBUF,)),
        ],
        compiler_params=pltpu.CompilerParams(needs_layout_passes=False),
    )(_sc_onehot)
    return k(xp, zeros)
